# baseline (speedup 1.0000x reference)
import functools

import jax
import jax.numpy as jnp
from jax import lax
from jax.experimental import pallas as pl
from jax.experimental.pallas import tpu as pltpu
from jax.experimental.pallas import tpu_sc as plsc

_NW = 32
_HL = 2
_D = 4


def kernel(x):
    B, Cr3, H, W, Z = x.shape
    C = Cr3 // 8
    WZ = W * Z
    SLAB = 4 * WZ
    QC = Cr3 // 4
    HG = H // _HL
    U = B * QC * HG
    T = U // _NW

    x4 = x.reshape(B, QC, 4, H, WZ)
    mesh = plsc.VectorSubcoreMesh(core_axis_name="c", subcore_axis_name="s")

    in_scratch = [pltpu.VMEM((4, _HL, WZ), jnp.float32) for _ in range(_D)]
    out_scratch = [pltpu.VMEM((_HL, SLAB), jnp.float32) for _ in range(_D)]

    @functools.partial(
        pl.kernel,
        mesh=mesh,
        compiler_params=pltpu.CompilerParams(needs_layout_passes=False),
        out_type=jax.ShapeDtypeStruct((B, C, H, 2 * SLAB), jnp.float32),
        scratch_types=in_scratch + out_scratch + [
            pltpu.SemaphoreType.DMA((_D,)),
            pltpu.SemaphoreType.DMA((_D,)),
        ],
    )
    def k(x_hbm, o_hbm, *refs):
        in_bufs = refs[:_D]
        out_bufs = refs[_D:2 * _D]
        in_sems, out_sems = refs[2 * _D], refs[2 * _D + 1]
        wid = lax.axis_index("c") * 16 + lax.axis_index("s")
        iota2 = 2 * lax.iota(jnp.int32, 16)

        def decode(t):
            u = t * _NW + wid
            b = u // (QC * HG)
            r = u % (QC * HG)
            return b, r // HG, r % HG

        def start_in(t, slot):
            b, qc, hg = decode(t)
            pltpu.make_async_copy(
                x_hbm.at[b, qc, :, pl.ds(hg * _HL, _HL), :],
                in_bufs[slot], in_sems.at[slot]).start()

        def wait_in(slot):
            pltpu.make_async_copy(
                x_hbm.at[0, 0, :, pl.ds(0, _HL), :],
                in_bufs[slot], in_sems.at[slot]).wait()

        def start_out(t, slot):
            b, qc, hg = decode(t)
            pltpu.make_async_copy(
                out_bufs[slot],
                o_hbm.at[b, qc // 2, pl.ds(hg * _HL, _HL),
                         pl.ds((qc % 2) * SLAB, SLAB)],
                out_sems.at[slot]).start()

        def wait_out(slot):
            pltpu.make_async_copy(
                out_bufs[slot],
                o_hbm.at[0, 0, pl.ds(0, _HL), pl.ds(0, SLAB)],
                out_sems.at[slot]).wait()

        for s in range(_D):
            start_in(s, s)

        def body(it, carry):
            for slot in range(_D):
                t = it * _D + slot
                wait_in(slot)

                @pl.when(t >= _D)
                def _drain():
                    wait_out(slot)

                src = in_bufs[slot]
                dst = out_bufs[slot]
                for hl in range(_HL):
                    hl_idx = jnp.full((16,), hl, jnp.int32)
                    for m in range(4):
                        c0 = 64 * (m // 2) + (m % 2)
                        base0 = iota2 + c0

                        @plsc.parallel_loop(0, 2 * W, 1, unroll=8)
                        def _scan(r):
                            data = src[m, hl, pl.ds(r * 16, 16)]
                            base = 64 * r - 32 * (r & 1)
                            plsc.store_scatter(
                                dst, [hl_idx, base0 + base], data
                            )

                start_out(t, slot)

                @pl.when(t + _D < T)
                def _prefetch():
                    start_in(t + _D, slot)
            return carry

        lax.fori_loop(0, T // _D, body, 0)
        for s in range(_D):
            wait_out(s)

    out = k(x4)
    return out.reshape(B, C, 2 * H, 2 * W, 2 * Z)
